# Initial kernel scaffold; baseline (speedup 1.0000x reference)
import jax
import jax.numpy as jnp
from jax.experimental import pallas as pl


def kernel(atom_fea, nbr_fea, nbr_fea_idx, crystal_atom_idx, emb_W, emb_b, convs_W, convs_b, convs_g1, convs_be1, convs_g2, convs_be2, res_W, res_b, res_g1, res_be1, res_g2, res_be2, fc_W, fc_b, out_W, out_b):
    raise NotImplementedError("write your pallas kernel here")


# speedup vs baseline: 1.5342x; 1.5342x over previous
import functools

import jax
import jax.numpy as jnp
from jax import lax
from jax.experimental import pallas as pl
from jax.experimental.pallas import tpu as pltpu
from jax.experimental.pallas import tpu_sc as plsc

N, M, F, NBRF, H, N0, A = 10000, 32, 64, 16, 128, 200, 50
TWO_F = 2 * F
TILE = 2000
AT = N // TILE
EPS = 1e-5

NC, NS = 2, 16
NW = NC * NS
CHUNK = 128
NCH = 80
RPW = NCH * CHUNK
KFIRE = 8
NSUPER = NCH // KFIRE


def _softplus(x):
    return jnp.maximum(x, 0.0) + jnp.log(1.0 + jnp.exp(-jnp.abs(x)))


def _sigmoid(x):
    return 1.0 / (1.0 + jnp.exp(-x))


def _sc_gather(table, idx_pad):
    mesh = plsc.VectorSubcoreMesh(core_axis_name="c", subcore_axis_name="s")

    @functools.partial(
        pl.kernel,
        out_type=jax.ShapeDtypeStruct((NW, RPW, F), jnp.float32),
        mesh=mesh,
        scratch_types=[
            pltpu.VMEM((NCH, CHUNK), jnp.int32),
            pltpu.VMEM((KFIRE, CHUNK, F), jnp.float32),
            pltpu.SemaphoreType.DMA,
            pltpu.SemaphoreType.DMA,
        ],
        compiler_params=pltpu.CompilerParams(use_tc_tiling_on_sc=False),
    )
    def gk(table_hbm, idx_hbm, out_hbm, idx_v, gbuf, gsem, wsem):
        wid = lax.axis_index("s") * NC + lax.axis_index("c")
        pltpu.sync_copy(idx_hbm.at[wid], idx_v)
        for ss in range(NSUPER):
            gs = [
                pltpu.async_copy(
                    table_hbm.at[idx_v.at[ss * KFIRE + b]], gbuf.at[b], gsem
                )
                for b in range(KFIRE)
            ]
            for g in gs:
                g.wait()
            ws = [
                pltpu.async_copy(
                    gbuf.at[b],
                    out_hbm.at[wid, pl.ds((ss * KFIRE + b) * CHUNK, CHUNK)],
                    wsem,
                )
                for b in range(KFIRE)
            ]
            for w in ws:
                w.wait()

    return gk(table, idx_pad)


def _emb(atom_fea, emb_W, emb_b2):
    def body(a_ref, w_ref, b_ref, o_ref):
        o_ref[...] = (
            jnp.dot(a_ref[...], w_ref[...], preferred_element_type=jnp.float32)
            + b_ref[...]
        )

    return pl.pallas_call(
        body,
        grid=(AT,),
        in_specs=[
            pl.BlockSpec((TILE, 128), lambda a: (a, 0)),
            pl.BlockSpec((128, F), lambda a: (0, 0)),
            pl.BlockSpec((1, F), lambda a: (0, 0)),
        ],
        out_specs=pl.BlockSpec((TILE, F), lambda a: (a, 0)),
        out_shape=jax.ShapeDtypeStruct((N, F), jnp.float32),
    )(atom_fea, emb_W, emb_b2)


def _stats_pass(x, xgT, nbrT, W, b2):
    def body(x_ref, xg_ref, nbr_ref, w_ref, b_ref, st_ref, u_s):
        a = pl.program_id(0)
        j = pl.program_id(1)

        @pl.when(j == 0)
        def _():
            u_s[...] = (
                jnp.dot(x_ref[...], w_ref[:F, :], preferred_element_type=jnp.float32)
                + b_ref[...]
            )

        gated = (
            u_s[...]
            + jnp.dot(xg_ref[0], w_ref[F:TWO_F, :], preferred_element_type=jnp.float32)
            + jnp.dot(nbr_ref[0], w_ref[TWO_F:, :], preferred_element_type=jnp.float32)
        )

        @pl.when(jnp.logical_and(a == 0, j == 0))
        def _():
            st_ref[...] = jnp.zeros_like(st_ref)

        st_ref[0:1, :] += jnp.sum(gated, axis=0, keepdims=True)
        st_ref[1:2, :] += jnp.sum(gated * gated, axis=0, keepdims=True)

    return pl.pallas_call(
        body,
        grid=(AT, M),
        in_specs=[
            pl.BlockSpec((TILE, F), lambda a, j: (a, 0)),
            pl.BlockSpec((1, TILE, F), lambda a, j: (j, a, 0)),
            pl.BlockSpec((1, TILE, NBRF), lambda a, j: (j, a, 0)),
            pl.BlockSpec((TWO_F + NBRF, TWO_F), lambda a, j: (0, 0)),
            pl.BlockSpec((1, TWO_F), lambda a, j: (0, 0)),
        ],
        out_specs=pl.BlockSpec((8, TWO_F), lambda a, j: (0, 0)),
        out_shape=jax.ShapeDtypeStruct((8, TWO_F), jnp.float32),
        scratch_shapes=[pltpu.VMEM((TILE, TWO_F), jnp.float32)],
    )(x, xgT, nbrT, W, b2)


def _apply_pass(x, xgT, nbrT, W, b2, stats, g1, be1):
    def body(x_ref, xg_ref, nbr_ref, w_ref, b_ref, st_ref, g1_ref, be1_ref,
             sum_ref, st2_ref, u_s):
        a = pl.program_id(0)
        j = pl.program_id(1)

        @pl.when(j == 0)
        def _():
            u_s[...] = (
                jnp.dot(x_ref[...], w_ref[:F, :], preferred_element_type=jnp.float32)
                + b_ref[...]
            )

        gated = (
            u_s[...]
            + jnp.dot(xg_ref[0], w_ref[F:TWO_F, :], preferred_element_type=jnp.float32)
            + jnp.dot(nbr_ref[0], w_ref[TWO_F:, :], preferred_element_type=jnp.float32)
        )

        cnt = jnp.float32(N * M)
        mu = st_ref[0:1, :] / cnt
        var = st_ref[1:2, :] / cnt - mu * mu
        rstd = lax.rsqrt(var + EPS)
        scale = g1_ref[...] * rstd
        shift = be1_ref[...] - mu * scale
        gn = gated * scale + shift
        contrib = _sigmoid(gn[:, :F]) * _softplus(gn[:, F:])

        @pl.when(j == 0)
        def _():
            sum_ref[...] = contrib

        @pl.when(j > 0)
        def _():
            sum_ref[...] += contrib

        @pl.when(j == M - 1)
        def _():
            @pl.when(a == 0)
            def _():
                st2_ref[...] = jnp.zeros_like(st2_ref)

            s = sum_ref[...]
            st2_ref[0:1, :] += jnp.sum(s, axis=0, keepdims=True)
            st2_ref[1:2, :] += jnp.sum(s * s, axis=0, keepdims=True)

    return pl.pallas_call(
        body,
        grid=(AT, M),
        in_specs=[
            pl.BlockSpec((TILE, F), lambda a, j: (a, 0)),
            pl.BlockSpec((1, TILE, F), lambda a, j: (j, a, 0)),
            pl.BlockSpec((1, TILE, NBRF), lambda a, j: (j, a, 0)),
            pl.BlockSpec((TWO_F + NBRF, TWO_F), lambda a, j: (0, 0)),
            pl.BlockSpec((1, TWO_F), lambda a, j: (0, 0)),
            pl.BlockSpec((8, TWO_F), lambda a, j: (0, 0)),
            pl.BlockSpec((1, TWO_F), lambda a, j: (0, 0)),
            pl.BlockSpec((1, TWO_F), lambda a, j: (0, 0)),
        ],
        out_specs=[
            pl.BlockSpec((TILE, F), lambda a, j: (a, 0)),
            pl.BlockSpec((8, F), lambda a, j: (0, 0)),
        ],
        out_shape=[
            jax.ShapeDtypeStruct((N, F), jnp.float32),
            jax.ShapeDtypeStruct((8, F), jnp.float32),
        ],
        scratch_shapes=[pltpu.VMEM((TILE, TWO_F), jnp.float32)],
    )(x, xgT, nbrT, W, b2, stats, g1, be1)


def _post_pass(x, summed, st2, g2, be2):
    def body(x_ref, s_ref, st2_ref, g2_ref, be2_ref, o_ref):
        cnt = jnp.float32(N)
        mu = st2_ref[0:1, :] / cnt
        var = st2_ref[1:2, :] / cnt - mu * mu
        rstd = lax.rsqrt(var + EPS)
        scale = g2_ref[...] * rstd
        shift = be2_ref[...] - mu * scale
        o_ref[...] = _softplus(x_ref[...] + s_ref[...] * scale + shift)

    return pl.pallas_call(
        body,
        grid=(AT,),
        in_specs=[
            pl.BlockSpec((TILE, F), lambda a: (a, 0)),
            pl.BlockSpec((TILE, F), lambda a: (a, 0)),
            pl.BlockSpec((8, F), lambda a: (0, 0)),
            pl.BlockSpec((1, F), lambda a: (0, 0)),
            pl.BlockSpec((1, F), lambda a: (0, 0)),
        ],
        out_specs=pl.BlockSpec((TILE, F), lambda a: (a, 0)),
        out_shape=jax.ShapeDtypeStruct((N, F), jnp.float32),
    )(x, summed, st2, g2, be2)


def _finale(x1, summed, st2, g2, be2, fc_W, fc_b2, out_Wr, out_b2):
    CRYS = TILE // A

    def body(x1_ref, s_ref, st2_ref, g2_ref, be2_ref, fw_ref, fb_ref,
             ow_ref, ob_ref, o_ref):
        cnt = jnp.float32(N)
        mu = st2_ref[0:1, :] / cnt
        var = st2_ref[1:2, :] / cnt - mu * mu
        rstd = lax.rsqrt(var + EPS)
        scale = g2_ref[...] * rstd
        shift = be2_ref[...] - mu * scale
        conv_out = _softplus(x1_ref[...] + s_ref[...] * scale + shift)
        x2 = _softplus(conv_out + x1_ref[...])

        rowid = lax.broadcasted_iota(jnp.int32, (CRYS, TILE), 0)
        colid = lax.broadcasted_iota(jnp.int32, (CRYS, TILE), 1)
        P = jnp.where(colid // A == rowid, jnp.float32(1.0 / A), jnp.float32(0.0))
        pooled = jnp.dot(P, x2, preferred_element_type=jnp.float32)
        sp = _softplus(pooled)
        h = _softplus(
            jnp.dot(sp, fw_ref[...], preferred_element_type=jnp.float32)
            + fb_ref[...]
        )
        o_ref[...] = (
            jnp.sum(h * ow_ref[...], axis=1, keepdims=True) + ob_ref[...]
        )

    return pl.pallas_call(
        body,
        grid=(AT,),
        in_specs=[
            pl.BlockSpec((TILE, F), lambda a: (a, 0)),
            pl.BlockSpec((TILE, F), lambda a: (a, 0)),
            pl.BlockSpec((8, F), lambda a: (0, 0)),
            pl.BlockSpec((1, F), lambda a: (0, 0)),
            pl.BlockSpec((1, F), lambda a: (0, 0)),
            pl.BlockSpec((F, H), lambda a: (0, 0)),
            pl.BlockSpec((1, H), lambda a: (0, 0)),
            pl.BlockSpec((1, H), lambda a: (0, 0)),
            pl.BlockSpec((1, 1), lambda a: (0, 0)),
        ],
        out_specs=pl.BlockSpec((CRYS, 1), lambda a: (a, 0)),
        out_shape=jax.ShapeDtypeStruct((N0, 1), jnp.float32),
    )(x1, summed, st2, g2, be2, fc_W, fc_b2, out_Wr, out_b2)


def _conv(x_in, idx_pad, nbrT, W, b2, g1, be1):
    xgT = _sc_gather(x_in, idx_pad)
    stats = _stats_pass(x_in, xgT, nbrT, W, b2)
    return _apply_pass(x_in, xgT, nbrT, W, b2, stats, g1, be1)


def kernel(atom_fea, nbr_fea, nbr_fea_idx, crystal_atom_idx, emb_W, emb_b,
           convs_W, convs_b, convs_g1, convs_be1, convs_g2, convs_be2,
           res_W, res_b, res_g1, res_be1, res_g2, res_be2,
           fc_W, fc_b, out_W, out_b):
    idxT = jnp.transpose(nbr_fea_idx.astype(jnp.int32), (1, 0))
    idx_pad = jnp.pad(idxT, ((0, 0), (0, RPW - N))).reshape(NW, NCH, CHUNK)
    nbrT = jnp.transpose(nbr_fea, (1, 0, 2))

    r2 = lambda v: v.reshape(1, -1)

    x = _emb(atom_fea, emb_W, r2(emb_b))

    summed, st2 = _conv(x, idx_pad, nbrT, convs_W[2], r2(convs_b[2]),
                        r2(convs_g1[2]), r2(convs_be1[2]))
    x1 = _post_pass(x, summed, st2, r2(convs_g2[2]), r2(convs_be2[2]))

    summed_r, st2_r = _conv(x1, idx_pad, nbrT, res_W, r2(res_b),
                            r2(res_g1), r2(res_be1))

    return _finale(x1, summed_r, st2_r, r2(res_g2), r2(res_be2),
                   fc_W, r2(fc_b), r2(out_W), out_b.reshape(1, 1))


# speedup vs baseline: 1.5425x; 1.0055x over previous
import functools

import jax
import jax.numpy as jnp
from jax import lax
from jax.experimental import pallas as pl
from jax.experimental.pallas import tpu as pltpu
from jax.experimental.pallas import tpu_sc as plsc

N, M, F, NBRF, H, N0, A = 10000, 32, 64, 16, 128, 200, 50
TWO_F = 2 * F
TILE = 2000
AT = N // TILE
EPS = 1e-5

NC, NS = 2, 16
NW = NC * NS
CHUNK = 128
NCH = 80
RPW = NCH * CHUNK
KFIRE = 5
NBANK = 2
NSUPER = NCH // KFIRE


def _softplus(x):
    return jnp.maximum(x, 0.0) + jnp.log(1.0 + jnp.exp(-jnp.abs(x)))


def _sigmoid(x):
    return 1.0 / (1.0 + jnp.exp(-x))


def _sc_gather(table, idx_pad):
    mesh = plsc.VectorSubcoreMesh(core_axis_name="c", subcore_axis_name="s")

    @functools.partial(
        pl.kernel,
        out_type=jax.ShapeDtypeStruct((NW, RPW, F), jnp.float32),
        mesh=mesh,
        scratch_types=[
            pltpu.VMEM((NCH, CHUNK), jnp.int32),
            pltpu.VMEM((NBANK, KFIRE, CHUNK, F), jnp.float32),
            pltpu.SemaphoreType.DMA,
            pltpu.SemaphoreType.DMA,
        ],
        compiler_params=pltpu.CompilerParams(use_tc_tiling_on_sc=False),
    )
    def gk(table_hbm, idx_hbm, out_hbm, idx_v, gbuf, gsem, wsem):
        wid = lax.axis_index("s") * NC + lax.axis_index("c")
        pltpu.sync_copy(idx_hbm.at[wid], idx_v)
        pend = [None] * NBANK
        for ss in range(NSUPER):
            bank = ss % NBANK
            if pend[bank] is not None:
                for w in pend[bank]:
                    w.wait()
            gs = [
                pltpu.async_copy(
                    table_hbm.at[idx_v.at[ss * KFIRE + b]], gbuf.at[bank, b], gsem
                )
                for b in range(KFIRE)
            ]
            for g in gs:
                g.wait()
            pend[bank] = [
                pltpu.async_copy(
                    gbuf.at[bank, b],
                    out_hbm.at[wid, pl.ds((ss * KFIRE + b) * CHUNK, CHUNK)],
                    wsem,
                )
                for b in range(KFIRE)
            ]
        for bank in range(NBANK):
            if pend[bank] is not None:
                for w in pend[bank]:
                    w.wait()

    return gk(table, idx_pad)


def _emb(atom_fea, emb_W, emb_b2):
    def body(a_ref, w_ref, b_ref, o_ref):
        o_ref[...] = (
            jnp.dot(a_ref[...], w_ref[...], preferred_element_type=jnp.float32)
            + b_ref[...]
        )

    return pl.pallas_call(
        body,
        grid=(AT,),
        in_specs=[
            pl.BlockSpec((TILE, 128), lambda a: (a, 0)),
            pl.BlockSpec((128, F), lambda a: (0, 0)),
            pl.BlockSpec((1, F), lambda a: (0, 0)),
        ],
        out_specs=pl.BlockSpec((TILE, F), lambda a: (a, 0)),
        out_shape=jax.ShapeDtypeStruct((N, F), jnp.float32),
    )(atom_fea, emb_W, emb_b2)


def _stats_pass(x, xgT, nbrT, W, b2):
    def body(x_ref, xg_ref, nbr_ref, w_ref, b_ref, st_ref, u_s):
        a = pl.program_id(0)
        j = pl.program_id(1)

        @pl.when(j == 0)
        def _():
            u_s[...] = (
                jnp.dot(x_ref[...], w_ref[:F, :], preferred_element_type=jnp.float32)
                + b_ref[...]
            )

        gated = (
            u_s[...]
            + jnp.dot(xg_ref[0], w_ref[F:TWO_F, :], preferred_element_type=jnp.float32)
            + jnp.dot(nbr_ref[0], w_ref[TWO_F:, :], preferred_element_type=jnp.float32)
        )

        @pl.when(jnp.logical_and(a == 0, j == 0))
        def _():
            st_ref[...] = jnp.zeros_like(st_ref)

        st_ref[0:1, :] += jnp.sum(gated, axis=0, keepdims=True)
        st_ref[8:9, :] += jnp.sum(gated * gated, axis=0, keepdims=True)

    return pl.pallas_call(
        body,
        grid=(AT, M),
        in_specs=[
            pl.BlockSpec((TILE, F), lambda a, j: (a, 0)),
            pl.BlockSpec((1, TILE, F), lambda a, j: (j, a, 0)),
            pl.BlockSpec((1, TILE, NBRF), lambda a, j: (j, a, 0)),
            pl.BlockSpec((TWO_F + NBRF, TWO_F), lambda a, j: (0, 0)),
            pl.BlockSpec((1, TWO_F), lambda a, j: (0, 0)),
        ],
        out_specs=pl.BlockSpec((16, TWO_F), lambda a, j: (0, 0)),
        out_shape=jax.ShapeDtypeStruct((16, TWO_F), jnp.float32),
        scratch_shapes=[pltpu.VMEM((TILE, TWO_F), jnp.float32)],
    )(x, xgT, nbrT, W, b2)


def _apply_pass(x, xgT, nbrT, W, b2, stats, g1, be1):
    def body(x_ref, xg_ref, nbr_ref, w_ref, b_ref, st_ref, g1_ref, be1_ref,
             sum_ref, st2_ref, u_s, w_s):
        a = pl.program_id(0)
        j = pl.program_id(1)

        @pl.when(jnp.logical_and(a == 0, j == 0))
        def _():
            cnt = jnp.float32(N * M)
            tot = (st_ref[0:1, :] + st_ref[1:2, :] + st_ref[2:3, :]
                   + st_ref[3:4, :] + st_ref[4:5, :] + st_ref[5:6, :]
                   + st_ref[6:7, :] + st_ref[7:8, :])
            tot2 = (st_ref[8:9, :] + st_ref[9:10, :] + st_ref[10:11, :]
                    + st_ref[11:12, :] + st_ref[12:13, :] + st_ref[13:14, :]
                    + st_ref[14:15, :] + st_ref[15:16, :])
            mu = tot / cnt
            var = tot2 / cnt - mu * mu
            rstd = lax.rsqrt(var + EPS)
            scale = g1_ref[...] * rstd
            shift = be1_ref[...] - mu * scale
            w_s[0:TWO_F + NBRF, :] = w_ref[...] * scale
            w_s[TWO_F + NBRF:TWO_F + NBRF + 1, :] = b_ref[...] * scale + shift

        @pl.when(j == 0)
        def _():
            u_s[...] = (
                jnp.dot(x_ref[...], w_s[:F, :], preferred_element_type=jnp.float32)
                + w_s[TWO_F + NBRF:TWO_F + NBRF + 1, :]
            )

        gn = (
            u_s[...]
            + jnp.dot(xg_ref[0], w_s[F:TWO_F, :], preferred_element_type=jnp.float32)
            + jnp.dot(nbr_ref[0], w_s[TWO_F:TWO_F + NBRF, :], preferred_element_type=jnp.float32)
        )
        contrib = _sigmoid(gn[:, :F]) * _softplus(gn[:, F:])

        @pl.when(j == 0)
        def _():
            sum_ref[...] = contrib

        @pl.when(j > 0)
        def _():
            sum_ref[...] += contrib

        @pl.when(j == M - 1)
        def _():
            @pl.when(a == 0)
            def _():
                st2_ref[...] = jnp.zeros_like(st2_ref)

            s = sum_ref[...]
            st2_ref[0:1, :] += jnp.sum(s, axis=0, keepdims=True)
            st2_ref[1:2, :] += jnp.sum(s * s, axis=0, keepdims=True)

    return pl.pallas_call(
        body,
        grid=(AT, M),
        in_specs=[
            pl.BlockSpec((TILE, F), lambda a, j: (a, 0)),
            pl.BlockSpec((1, TILE, F), lambda a, j: (j, a, 0)),
            pl.BlockSpec((1, TILE, NBRF), lambda a, j: (j, a, 0)),
            pl.BlockSpec((TWO_F + NBRF, TWO_F), lambda a, j: (0, 0)),
            pl.BlockSpec((1, TWO_F), lambda a, j: (0, 0)),
            pl.BlockSpec((16, TWO_F), lambda a, j: (0, 0)),
            pl.BlockSpec((1, TWO_F), lambda a, j: (0, 0)),
            pl.BlockSpec((1, TWO_F), lambda a, j: (0, 0)),
        ],
        out_specs=[
            pl.BlockSpec((TILE, F), lambda a, j: (a, 0)),
            pl.BlockSpec((8, F), lambda a, j: (0, 0)),
        ],
        out_shape=[
            jax.ShapeDtypeStruct((N, F), jnp.float32),
            jax.ShapeDtypeStruct((8, F), jnp.float32),
        ],
        scratch_shapes=[
            pltpu.VMEM((TILE, TWO_F), jnp.float32),
            pltpu.VMEM((TWO_F + NBRF + 8, TWO_F), jnp.float32),
        ],
    )(x, xgT, nbrT, W, b2, stats, g1, be1)


def _post_pass(x, summed, st2, g2, be2):
    def body(x_ref, s_ref, st2_ref, g2_ref, be2_ref, o_ref):
        cnt = jnp.float32(N)
        mu = st2_ref[0:1, :] / cnt
        var = st2_ref[1:2, :] / cnt - mu * mu
        rstd = lax.rsqrt(var + EPS)
        scale = g2_ref[...] * rstd
        shift = be2_ref[...] - mu * scale
        o_ref[...] = _softplus(x_ref[...] + s_ref[...] * scale + shift)

    return pl.pallas_call(
        body,
        grid=(AT,),
        in_specs=[
            pl.BlockSpec((TILE, F), lambda a: (a, 0)),
            pl.BlockSpec((TILE, F), lambda a: (a, 0)),
            pl.BlockSpec((8, F), lambda a: (0, 0)),
            pl.BlockSpec((1, F), lambda a: (0, 0)),
            pl.BlockSpec((1, F), lambda a: (0, 0)),
        ],
        out_specs=pl.BlockSpec((TILE, F), lambda a: (a, 0)),
        out_shape=jax.ShapeDtypeStruct((N, F), jnp.float32),
    )(x, summed, st2, g2, be2)


def _finale(x1, summed, st2, g2, be2, fc_W, fc_b2, out_Wr, out_b2):
    FT = 2000
    FAT = N // FT
    CRYS = FT // A

    def body(x1_ref, s_ref, st2_ref, g2_ref, be2_ref, fw_ref, fb_ref,
             ow_ref, ob_ref, o_ref):
        cnt = jnp.float32(N)
        mu = st2_ref[0:1, :] / cnt
        var = st2_ref[1:2, :] / cnt - mu * mu
        rstd = lax.rsqrt(var + EPS)
        scale = g2_ref[...] * rstd
        shift = be2_ref[...] - mu * scale
        conv_out = _softplus(x1_ref[...] + s_ref[...] * scale + shift)
        x2 = _softplus(conv_out + x1_ref[...])

        rowid = lax.broadcasted_iota(jnp.int32, (CRYS, FT), 0)
        colid = lax.broadcasted_iota(jnp.int32, (CRYS, FT), 1)
        P = jnp.where(colid // A == rowid, jnp.float32(1.0 / A), jnp.float32(0.0))
        pooled = jnp.dot(P, x2, preferred_element_type=jnp.float32)
        sp = _softplus(pooled)
        h = _softplus(
            jnp.dot(sp, fw_ref[...], preferred_element_type=jnp.float32)
            + fb_ref[...]
        )
        o_ref[...] = (
            jnp.sum(h * ow_ref[...], axis=1, keepdims=True) + ob_ref[...]
        )

    return pl.pallas_call(
        body,
        grid=(FAT,),
        in_specs=[
            pl.BlockSpec((FT, F), lambda a: (a, 0)),
            pl.BlockSpec((FT, F), lambda a: (a, 0)),
            pl.BlockSpec((8, F), lambda a: (0, 0)),
            pl.BlockSpec((1, F), lambda a: (0, 0)),
            pl.BlockSpec((1, F), lambda a: (0, 0)),
            pl.BlockSpec((F, H), lambda a: (0, 0)),
            pl.BlockSpec((1, H), lambda a: (0, 0)),
            pl.BlockSpec((1, H), lambda a: (0, 0)),
            pl.BlockSpec((1, 1), lambda a: (0, 0)),
        ],
        out_specs=pl.BlockSpec((CRYS, 1), lambda a: (a, 0)),
        out_shape=jax.ShapeDtypeStruct((N0, 1), jnp.float32),
    )(x1, summed, st2, g2, be2, fc_W, fc_b2, out_Wr, out_b2)


def _conv(x_in, idx_pad, nbrT, W, b2, g1, be1):
    xgT = _sc_gather(x_in, idx_pad)
    stats = _stats_pass(x_in, xgT, nbrT, W, b2)
    return _apply_pass(x_in, xgT, nbrT, W, b2, stats, g1, be1)


def kernel(atom_fea, nbr_fea, nbr_fea_idx, crystal_atom_idx, emb_W, emb_b,
           convs_W, convs_b, convs_g1, convs_be1, convs_g2, convs_be2,
           res_W, res_b, res_g1, res_be1, res_g2, res_be2,
           fc_W, fc_b, out_W, out_b):
    idxT = jnp.transpose(nbr_fea_idx.astype(jnp.int32), (1, 0))
    idx_pad = jnp.pad(idxT, ((0, 0), (0, RPW - N))).reshape(NW, NCH, CHUNK)
    nbrT = jnp.transpose(nbr_fea, (1, 0, 2))

    r2 = lambda v: v.reshape(1, -1)

    x = _emb(atom_fea, emb_W, r2(emb_b))

    summed, st2 = _conv(x, idx_pad, nbrT, convs_W[2], r2(convs_b[2]),
                        r2(convs_g1[2]), r2(convs_be1[2]))
    x1 = _post_pass(x, summed, st2, r2(convs_g2[2]), r2(convs_be2[2]))

    summed_r, st2_r = _conv(x1, idx_pad, nbrT, res_W, r2(res_b),
                            r2(res_g1), r2(res_be1))

    return _finale(x1, summed_r, st2_r, r2(res_g2), r2(res_be2),
                   fc_W, r2(fc_b), r2(out_W), out_b.reshape(1, 1))
